# Initial kernel scaffold; baseline (speedup 1.0000x reference)
import jax
import jax.numpy as jnp
from jax.experimental import pallas as pl


def kernel(y, W_in, b_in, norm1_g, norm1_b, edge_W1, edge_b1, edge_W2, edge_b2, node_W1, node_b1, node_W2, node_b2, norm2_g, norm2_b, mlp_W1, mlp_b1, mlp_W2, mlp_b2, W_out, b_out):
    raise NotImplementedError("write your pallas kernel here")


# speedup vs baseline: 16.5568x; 16.5568x over previous
import functools

import numpy as np
import jax
import jax.numpy as jnp
from jax import lax
from jax.experimental import pallas as pl
from jax.experimental.pallas import tpu as pltpu
from jax.experimental.pallas import tpu_sc as plsc

B = 8
G = 16
NG = G ** 3
CD = 64
HD = 64
NB = 2
K = 16
SP = 2.0
N = B * NG
ROWS = K * N

T = 512
NT = N // T

NWORK = 32
RPW = ROWS // NWORK
CH = 512
NCH = RPW // CH
TW = 2 * HD


_GRAPH_CACHE = None


def _graph_tables():
    global _GRAPH_CACHE
    if _GRAPH_CACHE is None:
        r = np.arange(G, dtype=np.float32) * np.float32(SP)
        xx, yy, zz = np.meshgrid(r, r, r, indexing="ij")
        c = np.stack([xx.ravel(), yy.ravel(), zz.ravel()], axis=-1)
        d2 = ((c[:, None, :] - c[None, :, :]) ** 2).sum(-1).astype(np.float32)
        d2 = d2 + np.eye(NG, dtype=np.float32) * np.float32(1e10)
        nbr = np.argsort(d2, axis=1, kind="stable")[:, :K]
        dist = np.sqrt(((c[nbr] - c[:, None, :]) ** 2).sum(-1)).astype(np.float32)
        idx = nbr.T[:, None, :] + (np.arange(B, dtype=np.int64) * NG)[None, :, None]
        idx_flat = idx.reshape(-1).astype(np.int32)
        _GRAPH_CACHE = (idx_flat, dist.astype(np.float32))
    return _GRAPH_CACHE


def _silu(x):
    return x * jax.nn.sigmoid(x)


def _ln(x, g, b):
    m = jnp.mean(x, axis=-1, keepdims=True)
    v = jnp.mean((x - m) ** 2, axis=-1, keepdims=True)
    return (x - m) / jnp.sqrt(v + 1e-5) * g + b


def _sc_gather(table, idx):
    mesh = plsc.VectorSubcoreMesh(core_axis_name="c", subcore_axis_name="s")

    @functools.partial(
        pl.kernel,
        out_type=jax.ShapeDtypeStruct((ROWS, TW), jnp.float32),
        mesh=mesh,
        scratch_types=[
            pltpu.VMEM((CH,), jnp.int32),
            pltpu.VMEM((CH, TW), jnp.float32),
            pltpu.SemaphoreType.DMA,
        ],
    )
    def gather_kernel(table_hbm, idx_hbm, out_hbm, idx_v, rows_v, sem):
        wid = lax.axis_index("s") * 2 + lax.axis_index("c")
        base0 = wid * RPW

        def body(ci, carry):
            base = base0 + ci * CH
            pltpu.sync_copy(idx_hbm.at[pl.ds(base, CH)], idx_v)
            pltpu.async_copy(table_hbm.at[idx_v], rows_v, sem).wait()
            pltpu.sync_copy(rows_v, out_hbm.at[pl.ds(base, CH)])
            return carry

        lax.fori_loop(0, NCH, body, 0)

    return gather_kernel(table, idx)


def _dot(a, b):
    return jnp.dot(a, b, preferred_element_type=jnp.float32)


def _tc_in_kernel(y_ref, Win_ref, bin_ref, g1_ref, b1_ref, W1ab_ref, eb1_ref,
                  h_ref, hn_ref, A_ref, Bc_ref):
    h = _dot(y_ref[...], Win_ref[...]) + bin_ref[...]
    hn = _ln(h, g1_ref[...], b1_ref[...])
    AB = _dot(hn, W1ab_ref[...])
    h_ref[...] = h
    hn_ref[...] = hn
    A_ref[...] = jnp.concatenate([AB[:, :HD], jnp.zeros((T, HD), jnp.float32)],
                                 axis=1)
    Bc_ref[...] = AB[:, HD:] + eb1_ref[...]


def _edge_agg(G_ref, Bc, dist_ref, w1c, Wbig, eb2x4):
    acc = jnp.zeros(Bc.shape, jnp.float32)
    for j in range(4):
        cols = []
        for k4 in range(4):
            k = 4 * j + k4
            m1 = _silu(G_ref[k][:, :HD] + Bc + dist_ref[:, k:k + 1] * w1c)
            cols.append(m1)
        X = jnp.concatenate(cols, axis=1)
        Y = _silu(_dot(X, Wbig) + eb2x4)
        acc = acc + (Y[:, 0:HD] + Y[:, HD:2 * HD]
                     + Y[:, 2 * HD:3 * HD] + Y[:, 3 * HD:4 * HD])
    return acc * (1.0 / K)


def _block_core(G_ref, h, hn, Bc, dist_ref, w1c_ref, Wbig_ref, eb2x4_ref,
                nW1_ref, nb1_ref, nW2_ref, nb2_ref, g2_ref, b2_ref,
                mW1_ref, mb1_ref, mW2_ref, mb2_ref):
    agg = _edge_agg(G_ref, Bc, dist_ref, w1c_ref[...], Wbig_ref[...],
                    eb2x4_ref[...])
    nd = jnp.concatenate([hn, agg], axis=1)
    t = _silu(_dot(nd, nW1_ref[...]) + nb1_ref[...])
    h1 = h + hn + _dot(t, nW2_ref[...]) + nb2_ref[...]
    hn2 = _ln(h1, g2_ref[...], b2_ref[...])
    mo = _dot(_silu(_dot(hn2, mW1_ref[...]) + mb1_ref[...]), mW2_ref[...]) \
        + mb2_ref[...]
    return h1 + mo


def _tc_mid_kernel(G_ref, h_ref, hn_ref, Bc_ref, dist_ref, w1c_ref, Wbig_ref,
                   eb2x4_ref, nW1_ref, nb1_ref, nW2_ref, nb2_ref, g2_ref,
                   b2_ref, mW1_ref, mb1_ref, mW2_ref, mb2_ref,
                   g1n_ref, b1n_ref, W1abn_ref, eb1n_ref,
                   h_out, hn_out, A_out, Bc_out):
    h2 = _block_core(G_ref, h_ref[...], hn_ref[...], Bc_ref[...], dist_ref,
                     w1c_ref, Wbig_ref, eb2x4_ref, nW1_ref, nb1_ref, nW2_ref,
                     nb2_ref, g2_ref, b2_ref, mW1_ref, mb1_ref, mW2_ref,
                     mb2_ref)
    hn_n = _ln(h2, g1n_ref[...], b1n_ref[...])
    ABn = _dot(hn_n, W1abn_ref[...])
    h_out[...] = h2
    hn_out[...] = hn_n
    A_out[...] = jnp.concatenate([ABn[:, :HD],
                                  jnp.zeros((T, HD), jnp.float32)], axis=1)
    Bc_out[...] = ABn[:, HD:] + eb1n_ref[...]


def _tc_last_kernel(G_ref, h_ref, hn_ref, Bc_ref, dist_ref, w1c_ref, Wbig_ref,
                    eb2x4_ref, nW1_ref, nb1_ref, nW2_ref, nb2_ref, g2_ref,
                    b2_ref, mW1_ref, mb1_ref, mW2_ref, mb2_ref,
                    Wout_ref, bout_ref, out_ref):
    h2 = _block_core(G_ref, h_ref[...], hn_ref[...], Bc_ref[...], dist_ref,
                     w1c_ref, Wbig_ref, eb2x4_ref, nW1_ref, nb1_ref, nW2_ref,
                     nb2_ref, g2_ref, b2_ref, mW1_ref, mb1_ref, mW2_ref,
                     mb2_ref)
    out_ref[...] = _dot(h2, Wout_ref[...]) + bout_ref[...]


def _tile_spec():
    return pl.BlockSpec((T, HD), lambda t: (t, 0))


def _full_spec(shape):
    return pl.BlockSpec(shape, lambda t: tuple(0 for _ in shape))


def _hABc_outs():
    shapes = [jax.ShapeDtypeStruct((N, HD), jnp.float32),
              jax.ShapeDtypeStruct((N, HD), jnp.float32),
              jax.ShapeDtypeStruct((N, TW), jnp.float32),
              jax.ShapeDtypeStruct((N, HD), jnp.float32)]
    specs = [_tile_spec(), _tile_spec(),
             pl.BlockSpec((T, TW), lambda t: (t, 0)), _tile_spec()]
    return shapes, specs


def _tc_in(y2d, W_in, b_in, g1, b1, W1ab, eb1):
    outs, ospecs = _hABc_outs()
    return pl.pallas_call(
        _tc_in_kernel,
        grid=(NT,),
        in_specs=[
            _tile_spec(),
            _full_spec((CD, HD)),
            _full_spec((1, HD)),
            _full_spec((1, HD)),
            _full_spec((1, HD)),
            _full_spec((HD, 2 * HD)),
            _full_spec((1, HD)),
        ],
        out_specs=ospecs,
        out_shape=outs,
    )(y2d, W_in, b_in, g1, b1, W1ab, eb1)


def _block_in_specs():
    return [
        pl.BlockSpec((K, T, TW), lambda t: (0, t, 0)),
        _tile_spec(),
        _tile_spec(),
        _tile_spec(),
        pl.BlockSpec((T, K), lambda t: (t % (NG // T), 0)),
        _full_spec((1, HD)),
        _full_spec((4 * HD, 4 * HD)),
        _full_spec((1, 4 * HD)),
        _full_spec((2 * HD, HD)),
        _full_spec((1, HD)),
        _full_spec((HD, HD)),
        _full_spec((1, HD)),
        _full_spec((1, HD)),
        _full_spec((1, HD)),
        _full_spec((HD, HD)),
        _full_spec((1, HD)),
        _full_spec((HD, HD)),
        _full_spec((1, HD)),
    ]


def _tc_mid(Gm, h, hn, Bc, dist, w1c, Wbig, eb2x4, nW1, nb1, nW2, nb2, g2, b2,
            mW1, mb1, mW2, mb2, g1n, b1n, W1abn, eb1n):
    outs, ospecs = _hABc_outs()
    return pl.pallas_call(
        _tc_mid_kernel,
        grid=(NT,),
        in_specs=_block_in_specs() + [
            _full_spec((1, HD)),
            _full_spec((1, HD)),
            _full_spec((HD, 2 * HD)),
            _full_spec((1, HD)),
        ],
        out_specs=ospecs,
        out_shape=outs,
    )(Gm, h, hn, Bc, dist, w1c, Wbig, eb2x4, nW1, nb1, nW2, nb2, g2, b2,
      mW1, mb1, mW2, mb2, g1n, b1n, W1abn, eb1n)


def _tc_last(Gm, h, hn, Bc, dist, w1c, Wbig, eb2x4, nW1, nb1, nW2, nb2, g2,
             b2, mW1, mb1, mW2, mb2, W_out, b_out):
    return pl.pallas_call(
        _tc_last_kernel,
        grid=(NT,),
        in_specs=_block_in_specs() + [
            _full_spec((HD, CD)),
            _full_spec((1, CD)),
        ],
        out_specs=pl.BlockSpec((T, CD), lambda t: (t, 0)),
        out_shape=jax.ShapeDtypeStruct((N, CD), jnp.float32),
    )(Gm, h, hn, Bc, dist, w1c, Wbig, eb2x4, nW1, nb1, nW2, nb2, g2, b2,
      mW1, mb1, mW2, mb2, W_out, b_out)


def kernel(y, W_in, b_in, norm1_g, norm1_b, edge_W1, edge_b1, edge_W2,
           edge_b2, node_W1, node_b1, node_W2, node_b2, norm2_g, norm2_b,
           mlp_W1, mlp_b1, mlp_W2, mlp_b2, W_out, b_out):
    idx_np, dist_np = _graph_tables()
    idx = jnp.asarray(idx_np)
    dist = jnp.asarray(dist_np)

    y2d = y.reshape(N, CD)
    row = lambda x: x.reshape(1, -1)

    W1ab = [jnp.concatenate([edge_W1[i][:HD, :], edge_W1[i][HD:2 * HD, :]],
                            axis=1) for i in range(NB)]
    w1c = [row(edge_W1[i][2 * HD, :]) for i in range(NB)]
    eye4 = jnp.eye(4, dtype=jnp.float32)
    Wbig = [jnp.kron(eye4, edge_W2[i]) for i in range(NB)]
    eb2x4 = [row(jnp.tile(edge_b2[i], 4)) for i in range(NB)]

    h, hn, A, Bc = _tc_in(y2d, W_in, row(b_in), row(norm1_g[0]),
                          row(norm1_b[0]), W1ab[0], row(edge_b1[0]))

    Gm = _sc_gather(A, idx).reshape(K, N, TW)
    h, hn, A, Bc = _tc_mid(
        Gm, h, hn, Bc, dist, w1c[0], Wbig[0], eb2x4[0], node_W1[0],
        row(node_b1[0]), node_W2[0], row(node_b2[0]), row(norm2_g[0]),
        row(norm2_b[0]), mlp_W1[0], row(mlp_b1[0]), mlp_W2[0],
        row(mlp_b2[0]), row(norm1_g[1]), row(norm1_b[1]), W1ab[1],
        row(edge_b1[1]))

    Gm = _sc_gather(A, idx).reshape(K, N, TW)
    out = _tc_last(
        Gm, h, hn, Bc, dist, w1c[1], Wbig[1], eb2x4[1], node_W1[1],
        row(node_b1[1]), node_W2[1], row(node_b2[1]), row(norm2_g[1]),
        row(norm2_b[1]), mlp_W1[1], row(mlp_b1[1]), mlp_W2[1],
        row(mlp_b2[1]), W_out, row(b_out))

    return out.reshape(B, NG, CD)
